# Initial kernel scaffold; baseline (speedup 1.0000x reference)
import jax
import jax.numpy as jnp
from jax.experimental import pallas as pl


def kernel(des, num, cat, W_des, b_des, W_num, b_num, W_cat, b_cat, W_inp, b_inp, W_proj, b_proj, att_src_f, att_dst_f, att_src_g, att_dst_g, W_k, b_k, q_sem, W_h1, b_h1, W_h2, b_h2, edge_follows, edge_friend):
    raise NotImplementedError("write your pallas kernel here")


# speedup vs baseline: 6.2917x; 6.2917x over previous
import dataclasses
import functools

import jax
import jax.numpy as jnp
from jax import lax
from jax.experimental import pallas as pl
from jax.experimental.pallas import tpu as pltpu
from jax.experimental.pallas import tpu_sc as plsc

_N = 10000
_E = 160000
_EP = 163840
_C = 256
_Q = 64
_NSUB = 16
_EPW = _EP // _NSUB
_NPAD = 10240
_OPAD = 10112
_BN = 2000
_CA = 2048
_CB = 512

_F32 = jnp.float32


def _lrelu(v, s):
    return jnp.maximum(v, s * v)


_GATHER_DN = lax.GatherDimensionNumbers(
    offset_dims=(), collapsed_slice_dims=(0,), start_index_map=(0,))


def _lane_gather(vec16, idx16):
    return lax.gather(vec16, idx16[:, None], _GATHER_DN, slice_sizes=(1,),
                      mode=lax.GatherScatterMode.PROMISE_IN_BOUNDS)


def _input_body(des, num, cat, Wd, bd, Wn, bn, Wc, bc, Wi, bi, x_out):
    dd = _lrelu(jnp.dot(des[...], Wd[...], preferred_element_type=_F32) + bd[...], 0.01)
    nn = _lrelu(jnp.dot(num[...], Wn[...], preferred_element_type=_F32) + bn[...], 0.01)
    cc = _lrelu(jnp.dot(cat[...], Wc[...], preferred_element_type=_F32) + bc[...], 0.01)
    xx = jnp.concatenate([dd, nn, cc], axis=1)
    x_out[...] = _lrelu(jnp.dot(xx, Wi[...], preferred_element_type=_F32) + bi[...], 0.01)


def _input_transform(des, num, cat, Wd, bd, Wn, bn, Wc, bc, Wi, bi):
    grid = (_N // _BN,)
    full = lambda r, c: pl.BlockSpec((r, c), lambda i: (0, 0))
    return pl.pallas_call(
        _input_body,
        grid=grid,
        in_specs=[
            pl.BlockSpec((_BN, 768), lambda i: (i, 0)),
            pl.BlockSpec((_BN, 6), lambda i: (i, 0)),
            pl.BlockSpec((_BN, 11), lambda i: (i, 0)),
            full(768, 128), full(1, 128),
            full(6, 64), full(1, 64),
            full(11, 64), full(1, 64),
            full(256, 256), full(1, 256),
        ],
        out_specs=pl.BlockSpec((_BN, _C), lambda i: (i, 0)),
        out_shape=jax.ShapeDtypeStruct((_N, _C), _F32),
    )(des, num, cat, Wd, bd, Wn, bn, Wc, bc, Wi, bi)


def _h_body(x, Wp, bp, att, hq0, hq1, hq2, hq3, scal):
    h = jnp.dot(x[...], Wp[...], preferred_element_type=_F32) + bp[...]
    for q, ref in enumerate((hq0, hq1, hq2, hq3)):
        ref[...] = h[:, q * _Q:(q + 1) * _Q]
    scal[...] = jnp.dot(h, att[...], preferred_element_type=_F32)


def _h_project(x, Wp, bp, att):
    grid = (_N // _BN,)
    full = lambda r, c: pl.BlockSpec((r, c), lambda i: (0, 0))
    qspec = pl.BlockSpec((_BN, _Q), lambda i: (i, 0))
    qshape = jax.ShapeDtypeStruct((_N, _Q), _F32)
    return pl.pallas_call(
        _h_body,
        grid=grid,
        in_specs=[
            pl.BlockSpec((_BN, _C), lambda i: (i, 0)),
            full(_C, _C), full(1, _C), full(_C, 4),
        ],
        out_specs=[qspec, qspec, qspec, qspec,
                   pl.BlockSpec((_BN, 4), lambda i: (i, 0))],
        out_shape=[qshape, qshape, qshape, qshape,
                   jax.ShapeDtypeStruct((_N, 4), _F32)],
    )(x, Wp, bp, att)


def _k1_body(u_ref, inv_ref, Wk, bk, o_ref, ksem_ref):
    t = pl.program_id(0)
    i = pl.program_id(1)
    oo = jnp.concatenate([u_ref[0, 0], u_ref[0, 1], u_ref[0, 2], u_ref[0, 3]],
                         axis=1)
    oo = jnp.maximum(oo, 0.0) * inv_ref[0]
    o_ref[0] = oo
    kk = jnp.tanh(jnp.dot(oo, Wk[...], preferred_element_type=_F32) + bk[...])
    part = jnp.sum(kk, axis=0, keepdims=True)

    @pl.when(i == 0)
    def _():
        ksem_ref[pl.ds(t, 1), :] = part

    @pl.when(i > 0)
    def _():
        ksem_ref[pl.ds(t, 1), :] = ksem_ref[pl.ds(t, 1), :] + part


def _k1(u, invr, Wk, bk):
    grid = (2, _N // _BN)
    full = lambda r, c: pl.BlockSpec((r, c), lambda t, i: (0, 0))
    return pl.pallas_call(
        _k1_body,
        grid=grid,
        in_specs=[
            pl.BlockSpec((1, 4, _BN, _Q), lambda t, i: (t, 0, i, 0)),
            pl.BlockSpec((1, _BN, 1), lambda t, i: (t, i, 0)),
            full(_C, _C), full(1, _C),
        ],
        out_specs=[
            pl.BlockSpec((1, _BN, _C), lambda t, i: (t, i, 0)),
            pl.BlockSpec((2, _C), lambda t, i: (0, 0)),
        ],
        out_shape=[
            jax.ShapeDtypeStruct((2, _N, _C), _F32),
            jax.ShapeDtypeStruct((2, _C), _F32),
        ],
    )(u, invr, Wk, bk)


def _semantic_attn(ksem_ref, q_ref):
    k0 = jnp.sum(ksem_ref[0:1, :] * q_ref[...]) / _N
    k1v = jnp.sum(ksem_ref[1:2, :] * q_ref[...]) / _N
    m = jnp.maximum(k0, k1v)
    e0 = jnp.exp(k0 - m)
    e1 = jnp.exp(k1v - m)
    a0 = e0 / (e0 + e1)
    return a0, 1.0 - a0


def _k2_body(o_ref, ksem_ref, q_ref, out_ref):
    a0, a1 = _semantic_attn(ksem_ref, q_ref)
    out_ref[...] = a0 * o_ref[0] + a1 * o_ref[1]


def _k2f_body(o_ref, ksem_ref, q_ref, W1, b1, W2, b2, out_ref):
    a0, a1 = _semantic_attn(ksem_ref, q_ref)
    comb = a0 * o_ref[0] + a1 * o_ref[1]
    y = _lrelu(jnp.dot(comb, W1[...], preferred_element_type=_F32) + b1[...], 0.01)
    out_ref[...] = jnp.dot(y, W2[...], preferred_element_type=_F32) + b2[...]


def _k2(o, ksem, qv):
    grid = (_N // _BN,)
    full = lambda r, c: pl.BlockSpec((r, c), lambda i: (0, 0))
    return pl.pallas_call(
        _k2_body,
        grid=grid,
        in_specs=[
            pl.BlockSpec((2, _BN, _C), lambda i: (0, i, 0)),
            full(2, _C), full(1, _C),
        ],
        out_specs=pl.BlockSpec((_BN, _C), lambda i: (i, 0)),
        out_shape=jax.ShapeDtypeStruct((_N, _C), _F32),
    )(o, ksem, qv)


def _k2_final(o, ksem, qv, W1, b1, W2, b2):
    grid = (_N // _BN,)
    full = lambda r, c: pl.BlockSpec((r, c), lambda i: (0, 0))
    return pl.pallas_call(
        _k2f_body,
        grid=grid,
        in_specs=[
            pl.BlockSpec((2, _BN, _C), lambda i: (0, i, 0)),
            full(2, _C), full(1, _C),
            full(_C, _C), full(1, _C),
            full(_C, 2), full(1, 2),
        ],
        out_specs=pl.BlockSpec((_BN, 2), lambda i: (i, 0)),
        out_shape=jax.ShapeDtypeStruct((_N, 2), _F32),
    )(o, ksem, qv, W1, b1, W2, b2)


_MESH = plsc.VectorSubcoreMesh(core_axis_name="c", subcore_axis_name="s")
_SC_PARAMS = dataclasses.replace(pltpu.CompilerParams(),
                                 needs_layout_passes=False,
                                 use_tc_tiling_on_sc=False)


def _phase_a_body(srcf, dstf, srcg, dstg, scal, inv2, e2,
                  scal_v, idx_v, dst_v, e_v, sbuf, s_sh):
    c = lax.axis_index("c")
    sid = lax.axis_index("s")
    pltpu.sync_copy(scal, scal_v)

    @pl.loop(0, 640, step=16)
    def _(i):
        sbuf[pl.ds(i, 16)] = jnp.zeros((16,), _F32)

    pltpu.sync_copy(sbuf, s_sh.at[pl.ds(sid * 640, 640)])
    plsc.subcore_barrier()

    cols_s = jnp.zeros((16,), jnp.int32) + 2 * c
    cols_d = cols_s + 1
    base0 = sid * _EPW

    def run(src_hbm, dst_hbm, e2row):
        @pl.loop(0, _EPW, step=_CA)
        def _(off):
            pltpu.sync_copy(src_hbm.at[pl.ds(base0 + off, _CA)], idx_v)
            pltpu.sync_copy(dst_hbm.at[pl.ds(base0 + off, _CA)], dst_v)

            @pl.loop(0, _CA, step=16)
            def _(j):
                s16 = idx_v[pl.ds(j, 16)]
                d16 = dst_v[pl.ds(j, 16)]
                a = plsc.load_gather(scal_v, [s16 * 4 + cols_s])
                b = plsc.load_gather(scal_v, [d16 * 4 + cols_d])
                z = a + b
                e_v[pl.ds(j, 16)] = jnp.exp(jnp.maximum(z, 0.2 * z))

            pltpu.sync_copy(e_v, e2row.at[pl.ds(base0 + off, _CA)])
            pltpu.sync_copy(e_v, s_sh.at[dst_v], add=True)

    @pl.when(c == 0)
    def _():
        run(srcf, dstf, e2.at[0])

    @pl.when(c == 1)
    def _():
        run(srcg, dstg, e2.at[1])

    plsc.subcore_barrier()
    pltpu.sync_copy(s_sh.at[pl.ds(sid * 640, 640)], sbuf)

    @pl.loop(0, 640, step=16)
    def _(i):
        v = sbuf[pl.ds(i, 16)]
        sbuf[pl.ds(i, 16)] = 1.0 / (v + 1e-16)

    pltpu.sync_copy(sbuf, inv2.at[c].at[pl.ds(sid * 640, 640)])


def _phase_a(srcf, dstf, srcg, dstg, scal):
    fn = pl.kernel(
        _phase_a_body,
        out_type=[
            jax.ShapeDtypeStruct((2, _NPAD), _F32),
            jax.ShapeDtypeStruct((2, _EP), _F32),
        ],
        mesh=_MESH,
        scratch_types=[
            pltpu.VMEM((_N * 4,), _F32),
            pltpu.VMEM((_CA,), jnp.int32),
            pltpu.VMEM((_CA,), jnp.int32),
            pltpu.VMEM((_CA,), _F32),
            pltpu.VMEM((640,), _F32),
            pltpu.VMEM_SHARED((_NPAD,), _F32),
        ],
        compiler_params=_SC_PARAMS,
    )
    return fn(srcf, dstf, srcg, dstg, scal)


def _phase_b_body(srcf, dstf, srcg, dstg, e2, hq0, hq1, hq2, hq3, u,
                  idx_v, dst_v, e_v, rows_v, z_v, o_sh):
    c = lax.axis_index("c")
    sid = lax.axis_index("s")

    @pl.loop(0, 128)
    def _(r):
        for k in range(_Q // 16):
            z_v[r, pl.ds(k * 16, 16)] = jnp.zeros((16,), _F32)

    def _sliced(fn):
        @pl.when(sid < 15)
        def _():
            for p in range(5):
                fn(sid * 640 + p * 128, 128)

        @pl.when(sid == 15)
        def _():
            for p in range(5):
                fn(9600 + p * 80, 80)

    base0 = sid * _EPW
    for t, (src_hbm, dst_hbm) in enumerate(((srcf, dstf), (srcg, dstg))):
        for r, quarters in enumerate(((hq0, hq1), (hq2, hq3))):
            _sliced(lambda rr, n: pltpu.sync_copy(z_v.at[pl.ds(0, n)],
                                                  o_sh.at[pl.ds(rr, n)]))
            plsc.subcore_barrier()

            @pl.loop(0, _EPW, step=_CB)
            def _(off, quarters=quarters, t=t):
                pltpu.sync_copy(src_hbm.at[pl.ds(base0 + off, _CB)], idx_v)
                pltpu.sync_copy(dst_hbm.at[pl.ds(base0 + off, _CB)], dst_v)
                pltpu.sync_copy(e2.at[t].at[pl.ds(base0 + off, _CB)], e_v)

                @pl.when(c == 0)
                def _():
                    pltpu.sync_copy(quarters[0].at[idx_v], rows_v)

                @pl.when(c == 1)
                def _():
                    pltpu.sync_copy(quarters[1].at[idx_v], rows_v)

                @pl.loop(0, _CB, step=16)
                def _(g):
                    ev16 = e_v[pl.ds(g, 16)]
                    for j in range(16):
                        wv = _lane_gather(ev16, jnp.zeros((16,), jnp.int32) + j)
                        for k in range(_Q // 16):
                            sl = (g + j, pl.ds(k * 16, 16))
                            rows_v[sl] = rows_v[sl] * wv

                pltpu.sync_copy(rows_v, o_sh.at[dst_v], add=True)

            plsc.subcore_barrier()

            def _flush(rr, n, t=t, r=r):
                pltpu.sync_copy(o_sh.at[pl.ds(rr, n)], rows_v.at[pl.ds(0, n)])
                pltpu.sync_copy(rows_v.at[pl.ds(0, n)],
                                u.at[t].at[2 * r + c].at[pl.ds(rr, n)])

            _sliced(_flush)


def _phase_b(srcf, dstf, srcg, dstg, e2, hq0, hq1, hq2, hq3):
    fn = pl.kernel(
        _phase_b_body,
        out_type=jax.ShapeDtypeStruct((2, 4, _N, _Q), _F32),
        mesh=_MESH,
        scratch_types=[
            pltpu.VMEM((_CB,), jnp.int32),
            pltpu.VMEM((_CB,), jnp.int32),
            pltpu.VMEM((_CB,), _F32),
            pltpu.VMEM((_CB, _Q), _F32),
            pltpu.VMEM((128, _Q), _F32),
            pltpu.VMEM_SHARED((_OPAD, _Q), _F32),
        ],
        compiler_params=_SC_PARAMS,
    )
    return fn(srcf, dstf, srcg, dstg, e2, hq0, hq1, hq2, hq3)


def kernel(des, num, cat, W_des, b_des, W_num, b_num, W_cat, b_cat, W_inp,
           b_inp, W_proj, b_proj, att_src_f, att_dst_f, att_src_g, att_dst_g,
           W_k, b_k, q_sem, W_h1, b_h1, W_h2, b_h2, edge_follows, edge_friend):
    row = lambda b: b.reshape(1, -1)
    att = jnp.stack([att_src_f, att_dst_f, att_src_g, att_dst_g], axis=1)
    pad_src = jnp.zeros((_EP - _E,), jnp.int32)
    pad_dst = _N + (jnp.arange(_EP - _E, dtype=jnp.int32) % 100)
    srcf = jnp.concatenate([edge_follows[0], pad_src])
    dstf = jnp.concatenate([edge_follows[1], pad_dst])
    srcg = jnp.concatenate([edge_friend[0], pad_src])
    dstg = jnp.concatenate([edge_friend[1], pad_dst])

    x = _input_transform(des, num, cat, W_des, row(b_des), W_num, row(b_num),
                         W_cat, row(b_cat), W_inp, row(b_inp))

    o = ksem = None
    for _layer in range(2):
        if _layer == 1:
            x = _k2(o, ksem, row(q_sem))
        hq0, hq1, hq2, hq3, scal = _h_project(x, W_proj, row(b_proj), att)
        inv2, e2 = _phase_a(srcf, dstf, srcg, dstg, scal.reshape(_N * 4))
        invr = inv2[:, :_N].reshape(2, _N, 1)
        u = _phase_b(srcf, dstf, srcg, dstg, e2, hq0, hq1, hq2, hq3)
        o, ksem = _k1(u, invr, W_k, row(b_k))

    return _k2_final(o, ksem, row(q_sem), W_h1[0], row(b_h1[0]),
                     W_h2[0], row(b_h2[0]))


# speedup vs baseline: 10.6758x; 1.6968x over previous
import dataclasses
import functools

import jax
import jax.numpy as jnp
from jax import lax
from jax.experimental import pallas as pl
from jax.experimental.pallas import tpu as pltpu
from jax.experimental.pallas import tpu_sc as plsc

_N = 10000
_E = 160000
_EP = 163840
_C = 256
_Q = 64
_NSUB = 16
_EPW = _EP // _NSUB
_NPAD = 10240
_OPAD = 10112
_BN = 2000
_CA = 2048
_CB = 512

_F32 = jnp.float32


def _lrelu(v, s):
    return jnp.maximum(v, s * v)


_GATHER_DN = lax.GatherDimensionNumbers(
    offset_dims=(), collapsed_slice_dims=(0,), start_index_map=(0,))


def _lane_gather(vec16, idx16):
    return lax.gather(vec16, idx16[:, None], _GATHER_DN, slice_sizes=(1,),
                      mode=lax.GatherScatterMode.PROMISE_IN_BOUNDS)


def _input_body(des, num, cat, Wd, bd, Wn, bn, Wc, bc, Wi, bi, x_out):
    dd = _lrelu(jnp.dot(des[...], Wd[...], preferred_element_type=_F32) + bd[...], 0.01)
    nn = _lrelu(jnp.dot(num[...], Wn[...], preferred_element_type=_F32) + bn[...], 0.01)
    cc = _lrelu(jnp.dot(cat[...], Wc[...], preferred_element_type=_F32) + bc[...], 0.01)
    xx = jnp.concatenate([dd, nn, cc], axis=1)
    x_out[...] = _lrelu(jnp.dot(xx, Wi[...], preferred_element_type=_F32) + bi[...], 0.01)


def _input_transform(des, num, cat, Wd, bd, Wn, bn, Wc, bc, Wi, bi):
    grid = (_N // _BN,)
    full = lambda r, c: pl.BlockSpec((r, c), lambda i: (0, 0))
    return pl.pallas_call(
        _input_body,
        grid=grid,
        in_specs=[
            pl.BlockSpec((_BN, 768), lambda i: (i, 0)),
            pl.BlockSpec((_BN, 6), lambda i: (i, 0)),
            pl.BlockSpec((_BN, 11), lambda i: (i, 0)),
            full(768, 128), full(1, 128),
            full(6, 64), full(1, 64),
            full(11, 64), full(1, 64),
            full(256, 256), full(1, 256),
        ],
        out_specs=pl.BlockSpec((_BN, _C), lambda i: (i, 0)),
        out_shape=jax.ShapeDtypeStruct((_N, _C), _F32),
    )(des, num, cat, Wd, bd, Wn, bn, Wc, bc, Wi, bi)


def _h_body(x, Wp, bp, att, hq0, hq1, hq2, hq3, scal):
    h = jnp.dot(x[...], Wp[...], preferred_element_type=_F32) + bp[...]
    for q, ref in enumerate((hq0, hq1, hq2, hq3)):
        ref[...] = h[:, q * _Q:(q + 1) * _Q]
    scal[...] = jnp.dot(h, att[...], preferred_element_type=_F32)


def _h_project(x, Wp, bp, att):
    grid = (_N // _BN,)
    full = lambda r, c: pl.BlockSpec((r, c), lambda i: (0, 0))
    qspec = pl.BlockSpec((_BN, _Q), lambda i: (i, 0))
    qshape = jax.ShapeDtypeStruct((_N, _Q), _F32)
    return pl.pallas_call(
        _h_body,
        grid=grid,
        in_specs=[
            pl.BlockSpec((_BN, _C), lambda i: (i, 0)),
            full(_C, _C), full(1, _C), full(_C, 4),
        ],
        out_specs=[qspec, qspec, qspec, qspec,
                   pl.BlockSpec((_BN, 4), lambda i: (i, 0))],
        out_shape=[qshape, qshape, qshape, qshape,
                   jax.ShapeDtypeStruct((_N, 4), _F32)],
    )(x, Wp, bp, att)


def _k1_body(u_ref, inv_ref, Wk, bk, o_ref, ksem_ref):
    t = pl.program_id(0)
    i = pl.program_id(1)
    oo = jnp.concatenate([u_ref[0, 0], u_ref[0, 1], u_ref[0, 2], u_ref[0, 3]],
                         axis=1)
    oo = jnp.maximum(oo, 0.0) * inv_ref[0]
    o_ref[0] = oo
    kk = jnp.tanh(jnp.dot(oo, Wk[...], preferred_element_type=_F32) + bk[...])
    part = jnp.sum(kk, axis=0, keepdims=True)

    @pl.when(i == 0)
    def _():
        ksem_ref[pl.ds(t, 1), :] = part

    @pl.when(i > 0)
    def _():
        ksem_ref[pl.ds(t, 1), :] = ksem_ref[pl.ds(t, 1), :] + part


def _k1(u, invr, Wk, bk):
    grid = (2, _N // _BN)
    full = lambda r, c: pl.BlockSpec((r, c), lambda t, i: (0, 0))
    return pl.pallas_call(
        _k1_body,
        grid=grid,
        in_specs=[
            pl.BlockSpec((1, 4, _BN, _Q), lambda t, i: (t, 0, i, 0)),
            pl.BlockSpec((1, _BN, 1), lambda t, i: (t, i, 0)),
            full(_C, _C), full(1, _C),
        ],
        out_specs=[
            pl.BlockSpec((1, _BN, _C), lambda t, i: (t, i, 0)),
            pl.BlockSpec((2, _C), lambda t, i: (0, 0)),
        ],
        out_shape=[
            jax.ShapeDtypeStruct((2, _N, _C), _F32),
            jax.ShapeDtypeStruct((2, _C), _F32),
        ],
    )(u, invr, Wk, bk)


def _semantic_attn(ksem_ref, q_ref):
    k0 = jnp.sum(ksem_ref[0:1, :] * q_ref[...]) / _N
    k1v = jnp.sum(ksem_ref[1:2, :] * q_ref[...]) / _N
    m = jnp.maximum(k0, k1v)
    e0 = jnp.exp(k0 - m)
    e1 = jnp.exp(k1v - m)
    a0 = e0 / (e0 + e1)
    return a0, 1.0 - a0


def _k2_body(o_ref, ksem_ref, q_ref, out_ref):
    a0, a1 = _semantic_attn(ksem_ref, q_ref)
    out_ref[...] = a0 * o_ref[0] + a1 * o_ref[1]


def _k2f_body(o_ref, ksem_ref, q_ref, W1, b1, W2, b2, out_ref):
    a0, a1 = _semantic_attn(ksem_ref, q_ref)
    comb = a0 * o_ref[0] + a1 * o_ref[1]
    y = _lrelu(jnp.dot(comb, W1[...], preferred_element_type=_F32) + b1[...], 0.01)
    out_ref[...] = jnp.dot(y, W2[...], preferred_element_type=_F32) + b2[...]


def _k2(o, ksem, qv):
    grid = (_N // _BN,)
    full = lambda r, c: pl.BlockSpec((r, c), lambda i: (0, 0))
    return pl.pallas_call(
        _k2_body,
        grid=grid,
        in_specs=[
            pl.BlockSpec((2, _BN, _C), lambda i: (0, i, 0)),
            full(2, _C), full(1, _C),
        ],
        out_specs=pl.BlockSpec((_BN, _C), lambda i: (i, 0)),
        out_shape=jax.ShapeDtypeStruct((_N, _C), _F32),
    )(o, ksem, qv)


def _k2_final(o, ksem, qv, W1, b1, W2, b2):
    grid = (_N // _BN,)
    full = lambda r, c: pl.BlockSpec((r, c), lambda i: (0, 0))
    return pl.pallas_call(
        _k2f_body,
        grid=grid,
        in_specs=[
            pl.BlockSpec((2, _BN, _C), lambda i: (0, i, 0)),
            full(2, _C), full(1, _C),
            full(_C, _C), full(1, _C),
            full(_C, 2), full(1, 2),
        ],
        out_specs=pl.BlockSpec((_BN, 2), lambda i: (i, 0)),
        out_shape=jax.ShapeDtypeStruct((_N, 2), _F32),
    )(o, ksem, qv, W1, b1, W2, b2)


_MESH = plsc.VectorSubcoreMesh(core_axis_name="c", subcore_axis_name="s")
_SC_PARAMS = dataclasses.replace(pltpu.CompilerParams(),
                                 needs_layout_passes=False,
                                 use_tc_tiling_on_sc=False)


def _phase_a_body(srcf, dstf, srcg, dstg, scal, inv2, e2,
                  scal_v, idx_v, dst_v, e_v, sbuf, s_sh):
    c = lax.axis_index("c")
    sid = lax.axis_index("s")
    pltpu.sync_copy(scal, scal_v)

    @pl.loop(0, 640, step=16)
    def _(i):
        sbuf[pl.ds(i, 16)] = jnp.zeros((16,), _F32)

    pltpu.sync_copy(sbuf, s_sh.at[pl.ds(sid * 640, 640)])
    plsc.subcore_barrier()

    cols_s = jnp.zeros((16,), jnp.int32) + 2 * c
    cols_d = cols_s + 1
    base0 = sid * _EPW

    def run(src_hbm, dst_hbm, e2row):
        @pl.loop(0, _EPW, step=_CA)
        def _(off):
            pltpu.sync_copy(src_hbm.at[pl.ds(base0 + off, _CA)], idx_v)
            pltpu.sync_copy(dst_hbm.at[pl.ds(base0 + off, _CA)], dst_v)

            @pl.loop(0, _CA, step=16)
            def _(j):
                s16 = idx_v[pl.ds(j, 16)]
                d16 = dst_v[pl.ds(j, 16)]
                a = plsc.load_gather(scal_v, [s16 * 4 + cols_s])
                b = plsc.load_gather(scal_v, [d16 * 4 + cols_d])
                z = a + b
                e_v[pl.ds(j, 16)] = jnp.exp(jnp.maximum(z, 0.2 * z))

            pltpu.sync_copy(e_v, e2row.at[pl.ds(base0 + off, _CA)])
            pltpu.sync_copy(e_v, s_sh.at[dst_v], add=True)

    @pl.when(c == 0)
    def _():
        run(srcf, dstf, e2.at[0])

    @pl.when(c == 1)
    def _():
        run(srcg, dstg, e2.at[1])

    plsc.subcore_barrier()
    pltpu.sync_copy(s_sh.at[pl.ds(sid * 640, 640)], sbuf)

    @pl.loop(0, 640, step=16)
    def _(i):
        v = sbuf[pl.ds(i, 16)]
        sbuf[pl.ds(i, 16)] = 1.0 / (v + 1e-16)

    pltpu.sync_copy(sbuf, inv2.at[c].at[pl.ds(sid * 640, 640)])


def _phase_a(srcf, dstf, srcg, dstg, scal):
    fn = pl.kernel(
        _phase_a_body,
        out_type=[
            jax.ShapeDtypeStruct((2, _NPAD), _F32),
            jax.ShapeDtypeStruct((2, _EP), _F32),
        ],
        mesh=_MESH,
        scratch_types=[
            pltpu.VMEM((_N * 4,), _F32),
            pltpu.VMEM((_CA,), jnp.int32),
            pltpu.VMEM((_CA,), jnp.int32),
            pltpu.VMEM((_CA,), _F32),
            pltpu.VMEM((640,), _F32),
            pltpu.VMEM_SHARED((_NPAD,), _F32),
        ],
        compiler_params=_SC_PARAMS,
    )
    return fn(srcf, dstf, srcg, dstg, scal)


def _phase_b_body(srcf, dstf, srcg, dstg, e2, hq0, hq1, hq2, hq3, u,
                  idx_v, dst_v, e_v, rows_v, z_v, gsem, o_sh):
    c = lax.axis_index("c")
    sid = lax.axis_index("s")

    @pl.loop(0, 128)
    def _(r):
        for k in range(_Q // 16):
            z_v[r, pl.ds(k * 16, 16)] = jnp.zeros((16,), _F32)

    def _sliced(fn):
        @pl.when(sid < 15)
        def _():
            for p in range(5):
                fn(sid * 640 + p * 128, 128)

        @pl.when(sid == 15)
        def _():
            for p in range(5):
                fn(9600 + p * 80, 80)

    base0 = sid * _EPW
    for t, (src_hbm, dst_hbm) in enumerate(((srcf, dstf), (srcg, dstg))):
        for quarters in ((hq0, hq1), (hq2, hq3)):

            def _load(b, off, t=t, src_hbm=src_hbm, dst_hbm=dst_hbm):
                pltpu.sync_copy(src_hbm.at[pl.ds(base0 + off, _CB)],
                                idx_v.at[b])
                pltpu.sync_copy(dst_hbm.at[pl.ds(base0 + off, _CB)],
                                dst_v.at[b])
                pltpu.sync_copy(e2.at[t].at[pl.ds(base0 + off, _CB)],
                                e_v.at[b])

            def _gather_start(b, quarters=quarters):
                @pl.when(c == 0)
                def _():
                    pltpu.async_copy(quarters[0].at[idx_v.at[b]],
                                     rows_v.at[b], gsem.at[b])

                @pl.when(c == 1)
                def _():
                    pltpu.async_copy(quarters[1].at[idx_v.at[b]],
                                     rows_v.at[b], gsem.at[b])

            def _gather_wait(b, quarters=quarters):
                pltpu.make_async_copy(quarters[0].at[idx_v.at[b]],
                                      rows_v.at[b], gsem.at[b]).wait()

            _sliced(lambda rr, n: pltpu.sync_copy(z_v.at[pl.ds(0, n)],
                                                  o_sh.at[pl.ds(rr, n)]))
            plsc.subcore_barrier()

            _load(0, 0)
            _gather_start(0)

            @pl.loop(0, _EPW, step=2 * _CB)
            def _(off):
                for b in range(2):
                    cur = off + b * _CB
                    nxt = cur + _CB

                    @pl.when(nxt < _EPW)
                    def _(b=b, nxt=nxt):
                        _load(1 - b, nxt)
                        _gather_start(1 - b)

                    _gather_wait(b)

                    @pl.loop(0, _CB, step=16)
                    def _(g, b=b):
                        ev16 = e_v[b, pl.ds(g, 16)]
                        for j in range(16):
                            wv = _lane_gather(ev16,
                                              jnp.zeros((16,), jnp.int32) + j)
                            for k in range(_Q // 16):
                                sl = (b, g + j, pl.ds(k * 16, 16))
                                rows_v[sl] = rows_v[sl] * wv

                    pltpu.sync_copy(rows_v.at[b], o_sh.at[dst_v.at[b]],
                                    add=True)

            plsc.subcore_barrier()

            q = 2 * (0 if quarters[0] is hq0 else 1) + c

            def _flush(rr, n, t=t, q=q):
                pltpu.sync_copy(o_sh.at[pl.ds(rr, n)], z_v.at[pl.ds(0, n)])
                pltpu.sync_copy(z_v.at[pl.ds(0, n)],
                                u.at[t].at[q].at[pl.ds(rr, n)])

            _sliced(_flush)

            @pl.loop(0, 128)
            def _(r):
                for k in range(_Q // 16):
                    z_v[r, pl.ds(k * 16, 16)] = jnp.zeros((16,), _F32)


def _phase_b(srcf, dstf, srcg, dstg, e2, hq0, hq1, hq2, hq3):
    fn = pl.kernel(
        _phase_b_body,
        out_type=jax.ShapeDtypeStruct((2, 4, _N, _Q), _F32),
        mesh=_MESH,
        scratch_types=[
            pltpu.VMEM((2, _CB), jnp.int32),
            pltpu.VMEM((2, _CB), jnp.int32),
            pltpu.VMEM((2, _CB), _F32),
            pltpu.VMEM((2, _CB, _Q), _F32),
            pltpu.VMEM((128, _Q), _F32),
            pltpu.SemaphoreType.DMA((2,)),
            pltpu.VMEM_SHARED((_OPAD, _Q), _F32),
        ],
        compiler_params=_SC_PARAMS,
    )
    return fn(srcf, dstf, srcg, dstg, e2, hq0, hq1, hq2, hq3)


def kernel(des, num, cat, W_des, b_des, W_num, b_num, W_cat, b_cat, W_inp,
           b_inp, W_proj, b_proj, att_src_f, att_dst_f, att_src_g, att_dst_g,
           W_k, b_k, q_sem, W_h1, b_h1, W_h2, b_h2, edge_follows, edge_friend):
    row = lambda b: b.reshape(1, -1)
    att = jnp.stack([att_src_f, att_dst_f, att_src_g, att_dst_g], axis=1)
    pad_src = jnp.zeros((_EP - _E,), jnp.int32)
    pad_dst = _N + (jnp.arange(_EP - _E, dtype=jnp.int32) % 100)
    srcf = jnp.concatenate([edge_follows[0], pad_src])
    dstf = jnp.concatenate([edge_follows[1], pad_dst])
    srcg = jnp.concatenate([edge_friend[0], pad_src])
    dstg = jnp.concatenate([edge_friend[1], pad_dst])

    x = _input_transform(des, num, cat, W_des, row(b_des), W_num, row(b_num),
                         W_cat, row(b_cat), W_inp, row(b_inp))

    def _layer(carry, _):
        x, _, _ = carry
        hq0, hq1, hq2, hq3, scal = _h_project(x, W_proj, row(b_proj), att)
        inv2, e2 = _phase_a(srcf, dstf, srcg, dstg, scal.reshape(_N * 4))
        invr = inv2[:, :_N].reshape(2, _N, 1)
        u = _phase_b(srcf, dstf, srcg, dstg, e2, hq0, hq1, hq2, hq3)
        o, ksem = _k1(u, invr, W_k, row(b_k))
        xn = _k2(o, ksem, row(q_sem))
        return (xn, o, ksem), None

    init = (x, jnp.zeros((2, _N, _C), _F32), jnp.zeros((2, _C), _F32))
    (x, o, ksem), _ = lax.scan(_layer, init, None, length=2)

    return _k2_final(o, ksem, row(q_sem), W_h1[0], row(b_h1[0]),
                     W_h2[0], row(b_h2[0]))


# speedup vs baseline: 10.6764x; 1.0001x over previous
import dataclasses
import functools

import jax
import jax.numpy as jnp
from jax import lax
from jax.experimental import pallas as pl
from jax.experimental.pallas import tpu as pltpu
from jax.experimental.pallas import tpu_sc as plsc

_N = 10000
_E = 160000
_EP = 163840
_C = 256
_Q = 64
_NSUB = 16
_EPW = _EP // _NSUB
_NPAD = 10240
_OPAD = 10112
_BN = 2000
_CA = 2048
_CB = 512

_F32 = jnp.float32


def _lrelu(v, s):
    return jnp.maximum(v, s * v)


_GATHER_DN = lax.GatherDimensionNumbers(
    offset_dims=(), collapsed_slice_dims=(0,), start_index_map=(0,))


def _lane_gather(vec16, idx16):
    return lax.gather(vec16, idx16[:, None], _GATHER_DN, slice_sizes=(1,),
                      mode=lax.GatherScatterMode.PROMISE_IN_BOUNDS)


def _input_body(des, num, cat, Wd, bd, Wn, bn, Wc, bc, Wi, bi, x_out):
    dd = _lrelu(jnp.dot(des[...], Wd[...], preferred_element_type=_F32) + bd[...], 0.01)
    nn = _lrelu(jnp.dot(num[...], Wn[...], preferred_element_type=_F32) + bn[...], 0.01)
    cc = _lrelu(jnp.dot(cat[...], Wc[...], preferred_element_type=_F32) + bc[...], 0.01)
    xx = jnp.concatenate([dd, nn, cc], axis=1)
    x_out[...] = _lrelu(jnp.dot(xx, Wi[...], preferred_element_type=_F32) + bi[...], 0.01)


def _input_transform(des, num, cat, Wd, bd, Wn, bn, Wc, bc, Wi, bi):
    grid = (_N // _BN,)
    full = lambda r, c: pl.BlockSpec((r, c), lambda i: (0, 0))
    return pl.pallas_call(
        _input_body,
        grid=grid,
        in_specs=[
            pl.BlockSpec((_BN, 768), lambda i: (i, 0)),
            pl.BlockSpec((_BN, 6), lambda i: (i, 0)),
            pl.BlockSpec((_BN, 11), lambda i: (i, 0)),
            full(768, 128), full(1, 128),
            full(6, 64), full(1, 64),
            full(11, 64), full(1, 64),
            full(256, 256), full(1, 256),
        ],
        out_specs=pl.BlockSpec((_BN, _C), lambda i: (i, 0)),
        out_shape=jax.ShapeDtypeStruct((_N, _C), _F32),
    )(des, num, cat, Wd, bd, Wn, bn, Wc, bc, Wi, bi)


def _h_body(x, Wp, bp, att, hq0, hq1, hq2, hq3, scal):
    h = jnp.dot(x[...], Wp[...], preferred_element_type=_F32) + bp[...]
    for q, ref in enumerate((hq0, hq1, hq2, hq3)):
        ref[...] = h[:, q * _Q:(q + 1) * _Q]
    scal[...] = jnp.dot(h, att[...], preferred_element_type=_F32)


def _h_project(x, Wp, bp, att):
    grid = (_N // _BN,)
    full = lambda r, c: pl.BlockSpec((r, c), lambda i: (0, 0))
    qspec = pl.BlockSpec((_BN, _Q), lambda i: (i, 0))
    qshape = jax.ShapeDtypeStruct((_N, _Q), _F32)
    return pl.pallas_call(
        _h_body,
        grid=grid,
        in_specs=[
            pl.BlockSpec((_BN, _C), lambda i: (i, 0)),
            full(_C, _C), full(1, _C), full(_C, 4),
        ],
        out_specs=[qspec, qspec, qspec, qspec,
                   pl.BlockSpec((_BN, 4), lambda i: (i, 0))],
        out_shape=[qshape, qshape, qshape, qshape,
                   jax.ShapeDtypeStruct((_N, 4), _F32)],
    )(x, Wp, bp, att)


def _k1_body(u_ref, inv_ref, Wk, bk, o_ref, ksem_ref):
    t = pl.program_id(0)
    i = pl.program_id(1)
    oo = jnp.concatenate([u_ref[0, 0], u_ref[0, 1], u_ref[0, 2], u_ref[0, 3]],
                         axis=1)
    oo = jnp.maximum(oo, 0.0) * inv_ref[0]
    o_ref[0] = oo
    kk = jnp.tanh(jnp.dot(oo, Wk[...], preferred_element_type=_F32) + bk[...])
    part = jnp.sum(kk, axis=0, keepdims=True)

    @pl.when(i == 0)
    def _():
        ksem_ref[pl.ds(t, 1), :] = part

    @pl.when(i > 0)
    def _():
        ksem_ref[pl.ds(t, 1), :] = ksem_ref[pl.ds(t, 1), :] + part


def _k1(u, invr, Wk, bk):
    grid = (2, _N // _BN)
    full = lambda r, c: pl.BlockSpec((r, c), lambda t, i: (0, 0))
    return pl.pallas_call(
        _k1_body,
        grid=grid,
        in_specs=[
            pl.BlockSpec((1, 4, _BN, _Q), lambda t, i: (t, 0, i, 0)),
            pl.BlockSpec((1, _BN, 1), lambda t, i: (t, i, 0)),
            full(_C, _C), full(1, _C),
        ],
        out_specs=[
            pl.BlockSpec((1, _BN, _C), lambda t, i: (t, i, 0)),
            pl.BlockSpec((2, _C), lambda t, i: (0, 0)),
        ],
        out_shape=[
            jax.ShapeDtypeStruct((2, _N, _C), _F32),
            jax.ShapeDtypeStruct((2, _C), _F32),
        ],
    )(u, invr, Wk, bk)


def _semantic_attn(ksem_ref, q_ref):
    k0 = jnp.sum(ksem_ref[0:1, :] * q_ref[...]) / _N
    k1v = jnp.sum(ksem_ref[1:2, :] * q_ref[...]) / _N
    m = jnp.maximum(k0, k1v)
    e0 = jnp.exp(k0 - m)
    e1 = jnp.exp(k1v - m)
    a0 = e0 / (e0 + e1)
    return a0, 1.0 - a0


def _k2_body(o_ref, ksem_ref, q_ref, out_ref):
    a0, a1 = _semantic_attn(ksem_ref, q_ref)
    out_ref[...] = a0 * o_ref[0] + a1 * o_ref[1]


def _k2f_body(o_ref, ksem_ref, q_ref, W1, b1, W2, b2, out_ref):
    a0, a1 = _semantic_attn(ksem_ref, q_ref)
    comb = a0 * o_ref[0] + a1 * o_ref[1]
    y = _lrelu(jnp.dot(comb, W1[...], preferred_element_type=_F32) + b1[...], 0.01)
    out_ref[...] = jnp.dot(y, W2[...], preferred_element_type=_F32) + b2[...]


def _k2(o, ksem, qv):
    grid = (_N // _BN,)
    full = lambda r, c: pl.BlockSpec((r, c), lambda i: (0, 0))
    return pl.pallas_call(
        _k2_body,
        grid=grid,
        in_specs=[
            pl.BlockSpec((2, _BN, _C), lambda i: (0, i, 0)),
            full(2, _C), full(1, _C),
        ],
        out_specs=pl.BlockSpec((_BN, _C), lambda i: (i, 0)),
        out_shape=jax.ShapeDtypeStruct((_N, _C), _F32),
    )(o, ksem, qv)


def _k2_final(o, ksem, qv, W1, b1, W2, b2):
    grid = (_N // _BN,)
    full = lambda r, c: pl.BlockSpec((r, c), lambda i: (0, 0))
    return pl.pallas_call(
        _k2f_body,
        grid=grid,
        in_specs=[
            pl.BlockSpec((2, _BN, _C), lambda i: (0, i, 0)),
            full(2, _C), full(1, _C),
            full(_C, _C), full(1, _C),
            full(_C, 2), full(1, 2),
        ],
        out_specs=pl.BlockSpec((_BN, 2), lambda i: (i, 0)),
        out_shape=jax.ShapeDtypeStruct((_N, 2), _F32),
    )(o, ksem, qv, W1, b1, W2, b2)


_MESH = plsc.VectorSubcoreMesh(core_axis_name="c", subcore_axis_name="s")
_SC_PARAMS = dataclasses.replace(pltpu.CompilerParams(),
                                 needs_layout_passes=False,
                                 use_tc_tiling_on_sc=False)


def _phase_a_body(srcf, dstf, srcg, dstg, scal, inv2, e2,
                  scal_v, idx_v, dst_v, e_v, sbuf, s_sh):
    c = lax.axis_index("c")
    sid = lax.axis_index("s")
    pltpu.sync_copy(scal, scal_v)

    @pl.loop(0, 640, step=16)
    def _(i):
        sbuf[pl.ds(i, 16)] = jnp.zeros((16,), _F32)

    pltpu.sync_copy(sbuf, s_sh.at[pl.ds(sid * 640, 640)])
    plsc.subcore_barrier()

    cols_s = jnp.zeros((16,), jnp.int32) + 2 * c
    cols_d = cols_s + 1
    base0 = sid * _EPW

    def run(src_hbm, dst_hbm, e2row):
        @pl.loop(0, _EPW, step=_CA)
        def _(off):
            pltpu.sync_copy(src_hbm.at[pl.ds(base0 + off, _CA)], idx_v)
            pltpu.sync_copy(dst_hbm.at[pl.ds(base0 + off, _CA)], dst_v)

            @pl.loop(0, _CA, step=16)
            def _(j):
                s16 = idx_v[pl.ds(j, 16)]
                d16 = dst_v[pl.ds(j, 16)]
                a = plsc.load_gather(scal_v, [s16 * 4 + cols_s])
                b = plsc.load_gather(scal_v, [d16 * 4 + cols_d])
                z = a + b
                e_v[pl.ds(j, 16)] = jnp.exp(jnp.maximum(z, 0.2 * z))

            pltpu.sync_copy(e_v, e2row.at[pl.ds(base0 + off, _CA)])
            pltpu.sync_copy(e_v, s_sh.at[dst_v], add=True)

    @pl.when(c == 0)
    def _():
        run(srcf, dstf, e2.at[0])

    @pl.when(c == 1)
    def _():
        run(srcg, dstg, e2.at[1])

    plsc.subcore_barrier()
    pltpu.sync_copy(s_sh.at[pl.ds(sid * 640, 640)], sbuf)

    @pl.loop(0, 640, step=16)
    def _(i):
        v = sbuf[pl.ds(i, 16)]
        sbuf[pl.ds(i, 16)] = 1.0 / (v + 1e-16)

    pltpu.sync_copy(sbuf, inv2.at[c].at[pl.ds(sid * 640, 640)])


def _phase_a(srcf, dstf, srcg, dstg, scal):
    fn = pl.kernel(
        _phase_a_body,
        out_type=[
            jax.ShapeDtypeStruct((2, _NPAD), _F32),
            jax.ShapeDtypeStruct((2, _EP), _F32),
        ],
        mesh=_MESH,
        scratch_types=[
            pltpu.VMEM((_N * 4,), _F32),
            pltpu.VMEM((_CA,), jnp.int32),
            pltpu.VMEM((_CA,), jnp.int32),
            pltpu.VMEM((_CA,), _F32),
            pltpu.VMEM((640,), _F32),
            pltpu.VMEM_SHARED((_NPAD,), _F32),
        ],
        compiler_params=_SC_PARAMS,
    )
    return fn(srcf, dstf, srcg, dstg, scal)


def _phase_b_body(srcf, dstf, srcg, dstg, e2, hq0, hq1, hq2, hq3, u,
                  idx_v, dst_v, e_v, rows_v, z_v, gsem, ssem, o_sh):
    c = lax.axis_index("c")
    sid = lax.axis_index("s")

    @pl.loop(0, 128)
    def _(r):
        for k in range(_Q // 16):
            z_v[r, pl.ds(k * 16, 16)] = jnp.zeros((16,), _F32)

    def _sliced(fn):
        @pl.when(sid < 15)
        def _():
            for p in range(5):
                fn(sid * 640 + p * 128, 128)

        @pl.when(sid == 15)
        def _():
            for p in range(5):
                fn(9600 + p * 80, 80)

    base0 = sid * _EPW
    for t, (src_hbm, dst_hbm) in enumerate(((srcf, dstf), (srcg, dstg))):
        for quarters in ((hq0, hq1), (hq2, hq3)):

            def _load(b, off, t=t, src_hbm=src_hbm, dst_hbm=dst_hbm):
                pltpu.sync_copy(src_hbm.at[pl.ds(base0 + off, _CB)],
                                idx_v.at[b])
                pltpu.sync_copy(dst_hbm.at[pl.ds(base0 + off, _CB)],
                                dst_v.at[b])
                pltpu.sync_copy(e2.at[t].at[pl.ds(base0 + off, _CB)],
                                e_v.at[b])

            def _gather_start(b, quarters=quarters):
                @pl.when(c == 0)
                def _():
                    pltpu.async_copy(quarters[0].at[idx_v.at[b]],
                                     rows_v.at[b], gsem.at[b])

                @pl.when(c == 1)
                def _():
                    pltpu.async_copy(quarters[1].at[idx_v.at[b]],
                                     rows_v.at[b], gsem.at[b])

            def _gather_wait(b, quarters=quarters):
                pltpu.make_async_copy(quarters[0].at[idx_v.at[b]],
                                      rows_v.at[b], gsem.at[b]).wait()

            def _scatter_wait(b):
                pltpu.make_async_copy(rows_v.at[b], o_sh.at[dst_v.at[b]],
                                      ssem.at[b]).wait()

            _sliced(lambda rr, n: pltpu.sync_copy(z_v.at[pl.ds(0, n)],
                                                  o_sh.at[pl.ds(rr, n)]))
            plsc.subcore_barrier()

            _load(0, 0)
            _gather_start(0)

            @pl.loop(0, _EPW, step=2 * _CB)
            def _(off):
                for b in range(2):
                    cur = off + b * _CB
                    nxt = cur + _CB

                    @pl.when(nxt < _EPW)
                    def _(b=b, nxt=nxt, cur=cur):
                        @pl.when(cur >= _CB)
                        def _():
                            _scatter_wait(1 - b)

                        _load(1 - b, nxt)
                        _gather_start(1 - b)

                    _gather_wait(b)

                    @pl.loop(0, _CB, step=16)
                    def _(g, b=b):
                        ev16 = e_v[b, pl.ds(g, 16)]
                        for j in range(16):
                            wv = _lane_gather(ev16,
                                              jnp.zeros((16,), jnp.int32) + j)
                            for k in range(_Q // 16):
                                sl = (b, g + j, pl.ds(k * 16, 16))
                                rows_v[sl] = rows_v[sl] * wv

                    pltpu.async_copy(rows_v.at[b], o_sh.at[dst_v.at[b]],
                                     ssem.at[b], add=True)

            _scatter_wait(0)
            _scatter_wait(1)
            plsc.subcore_barrier()

            q = 2 * (0 if quarters[0] is hq0 else 1) + c

            def _flush(rr, n, t=t, q=q):
                pltpu.sync_copy(o_sh.at[pl.ds(rr, n)], z_v.at[pl.ds(0, n)])
                pltpu.sync_copy(z_v.at[pl.ds(0, n)],
                                u.at[t].at[q].at[pl.ds(rr, n)])

            _sliced(_flush)

            @pl.loop(0, 128)
            def _(r):
                for k in range(_Q // 16):
                    z_v[r, pl.ds(k * 16, 16)] = jnp.zeros((16,), _F32)


def _phase_b(srcf, dstf, srcg, dstg, e2, hq0, hq1, hq2, hq3):
    fn = pl.kernel(
        _phase_b_body,
        out_type=jax.ShapeDtypeStruct((2, 4, _N, _Q), _F32),
        mesh=_MESH,
        scratch_types=[
            pltpu.VMEM((2, _CB), jnp.int32),
            pltpu.VMEM((2, _CB), jnp.int32),
            pltpu.VMEM((2, _CB), _F32),
            pltpu.VMEM((2, _CB, _Q), _F32),
            pltpu.VMEM((128, _Q), _F32),
            pltpu.SemaphoreType.DMA((2,)),
            pltpu.SemaphoreType.DMA((2,)),
            pltpu.VMEM_SHARED((_OPAD, _Q), _F32),
        ],
        compiler_params=_SC_PARAMS,
    )
    return fn(srcf, dstf, srcg, dstg, e2, hq0, hq1, hq2, hq3)


def kernel(des, num, cat, W_des, b_des, W_num, b_num, W_cat, b_cat, W_inp,
           b_inp, W_proj, b_proj, att_src_f, att_dst_f, att_src_g, att_dst_g,
           W_k, b_k, q_sem, W_h1, b_h1, W_h2, b_h2, edge_follows, edge_friend):
    row = lambda b: b.reshape(1, -1)
    att = jnp.stack([att_src_f, att_dst_f, att_src_g, att_dst_g], axis=1)
    pad_src = jnp.zeros((_EP - _E,), jnp.int32)
    pad_dst = _N + (jnp.arange(_EP - _E, dtype=jnp.int32) % 100)
    srcf = jnp.concatenate([edge_follows[0], pad_src])
    dstf = jnp.concatenate([edge_follows[1], pad_dst])
    srcg = jnp.concatenate([edge_friend[0], pad_src])
    dstg = jnp.concatenate([edge_friend[1], pad_dst])

    x = _input_transform(des, num, cat, W_des, row(b_des), W_num, row(b_num),
                         W_cat, row(b_cat), W_inp, row(b_inp))

    def _layer(carry, _):
        x, _, _ = carry
        hq0, hq1, hq2, hq3, scal = _h_project(x, W_proj, row(b_proj), att)
        inv2, e2 = _phase_a(srcf, dstf, srcg, dstg, scal.reshape(_N * 4))
        invr = inv2[:, :_N].reshape(2, _N, 1)
        u = _phase_b(srcf, dstf, srcg, dstg, e2, hq0, hq1, hq2, hq3)
        o, ksem = _k1(u, invr, W_k, row(b_k))
        xn = _k2(o, ksem, row(q_sem))
        return (xn, o, ksem), None

    init = (x, jnp.zeros((2, _N, _C), _F32), jnp.zeros((2, _C), _F32))
    (x, o, ksem), _ = lax.scan(_layer, init, None, length=2)

    return _k2_final(o, ksem, row(q_sem), W_h1[0], row(b_h1[0]),
                     W_h2[0], row(b_h2[0]))
